# baseline (speedup 1.0000x reference)
import functools

import jax
import jax.numpy as jnp
from jax import lax
from jax.experimental import pallas as pl
from jax.experimental.pallas import tpu as pltpu
from jax.experimental.pallas import tpu_sc as plsc

N_SC_FRAC_NUM, N_SC_FRAC_DEN = 2, 5
N_SC_QUANTUM = 64000
TC_BLK = 2000


def _segment_sum_sc(attr, idx3, num_segments, n_sc):
  N, F = attr.shape
  info = plsc.get_sparse_core_info()
  NC, NS, L = info.num_cores, info.num_subcores, info.num_lanes
  NW = NC * NS
  NF = F // L
  rows_per_w = n_sc // NW
  R = 80
  steps = rows_per_w // R
  NBUF = 4
  NG = R // L

  mesh = plsc.VectorSubcoreMesh(core_axis_name="c", subcore_axis_name="s")

  @functools.partial(
      pl.kernel,
      mesh=mesh,
      out_type=jax.ShapeDtypeStruct((NC, num_segments, F), jnp.float32),
      scratch_types=[
          pltpu.VMEM((steps, R), jnp.int32),
          pltpu.VMEM((NBUF, R, F), jnp.float32),
          pltpu.VMEM((num_segments, F), jnp.float32),
          pltpu.VMEM((num_segments,), jnp.int32),
          pltpu.VMEM_SHARED((num_segments, F), jnp.float32),
          pltpu.SemaphoreType.DMA((NBUF,)),
      ],
  )
  def k(attr_hbm, idx_hbm, out_hbm, idx_all, rows, acc_v, iota_v, acc_sh,
        gsem):
    cid = lax.axis_index("c")
    sid = lax.axis_index("s")
    wid = sid * NC + cid
    base = wid * rows_per_w

    def zrow(i, carry):
      for j in range(NF):
        acc_v[i, pl.ds(j * L, L)] = jnp.zeros((L,), jnp.float32)
      return carry
    lax.fori_loop(0, num_segments, zrow, 0)

    @pl.when(sid == 0)
    def _():
      pltpu.sync_copy(acc_v, acc_sh)

    for kk in range(num_segments // L):
      iota_v[pl.ds(kk * L, L)] = lax.iota(jnp.int32, L) + (kk * L)

    pltpu.sync_copy(idx_hbm.at[wid], idx_all)

    plsc.subcore_barrier()

    def gather(t, bi):
      return pltpu.async_copy(
          attr_hbm.at[pl.ds(base + t * R, R)], rows.at[bi], gsem.at[bi])

    def wait_gather(t, bi):
      pltpu.make_async_copy(
          attr_hbm.at[pl.ds(base + t * R, R)], rows.at[bi], gsem.at[bi]
      ).wait()

    for b in range(NBUF):
      gather(b, b)

    def body(t, carry):
      bi = t & (NBUF - 1)
      wait_gather(t, bi)
      for gi in range(NG):
        a = gi * L
        ids = idx_all[t, pl.ds(a, L)]
        s0 = ids[0]
        s15 = ids[L - 1]

        @pl.when(s0 == s15)
        def _():
          acc = tuple(rows[bi, a, pl.ds(j * L, L)] for j in range(NF))
          for r in range(1, L):
            vals = tuple(rows[bi, a + r, pl.ds(j * L, L)] for j in range(NF))
            acc = tuple(acc[j] + vals[j] for j in range(NF))
          for j in range(NF):
            plsc.addupdate(acc_v.at[s0, pl.ds(j * L, L)], acc[j])

        @pl.when(s0 != s15)
        def _():
          for r in range(L):
            s_r = ids[r]
            for j in range(NF):
              plsc.addupdate(acc_v.at[s_r, pl.ds(j * L, L)],
                             rows[bi, a + r, pl.ds(j * L, L)])

      @pl.when(t + NBUF < steps)
      def _():
        gather(t + NBUF, bi)
      return carry

    lax.fori_loop(0, steps, body, 0)

    pltpu.sync_copy(acc_v, acc_sh.at[iota_v], add=True)

    plsc.subcore_barrier()

    @pl.when(sid == 0)
    def _():
      pltpu.sync_copy(acc_sh, out_hbm.at[cid])

  return k(attr, idx3)


def _segment_sum_tc(attr, idx3, num_segments, n_sc):
  N, F = attr.shape
  nb = (N - n_sc) // TC_BLK
  off = n_sc // TC_BLK

  def body(ids_ref, attr_ref, out_ref):
    i = pl.program_id(0)

    @pl.when(i == 0)
    def _():
      out_ref[...] = jnp.zeros_like(out_ref)

    ids = ids_ref[0]
    s0 = ids_ref[0, 0, 0]
    s_last = ids_ref[0, 0, TC_BLK - 1]

    @pl.when(s0 == s_last)
    def _():
      colsum = jnp.sum(attr_ref[...], axis=0, keepdims=True)
      rows_iota = lax.broadcasted_iota(jnp.int32, (num_segments, 1), 0)
      mask = (rows_iota == s0).astype(jnp.float32)
      out_ref[...] += mask * colsum

    @pl.when(s0 != s_last)
    def _():
      iota_s = lax.broadcasted_iota(jnp.int32, (num_segments, TC_BLK), 0)
      onehot = (iota_s == ids).astype(jnp.float32)
      out_ref[...] += jnp.dot(onehot, attr_ref[...],
                              preferred_element_type=jnp.float32)

  return pl.pallas_call(
      body,
      grid=(nb,),
      in_specs=[
          pl.BlockSpec((1, 1, TC_BLK), lambda i: (i + off, 0, 0)),
          pl.BlockSpec((TC_BLK, F), lambda i: (i + off, 0)),
      ],
      out_specs=pl.BlockSpec((num_segments, F), lambda i: (0, 0)),
      out_shape=jax.ShapeDtypeStruct((num_segments, F), jnp.float32),
  )(idx3.reshape(N // TC_BLK, 1, TC_BLK), attr)


def kernel(reference, attr, batch_index):
  num_segments = reference.shape[0]
  N = attr.shape[0]
  n_sc = max(N_SC_QUANTUM,
             (N * N_SC_FRAC_NUM // N_SC_FRAC_DEN // N_SC_QUANTUM)
             * N_SC_QUANTUM)
  idx = batch_index.astype(jnp.int32)
  info = plsc.get_sparse_core_info()
  NW = info.num_cores * info.num_subcores
  rows_per_w = n_sc // NW
  sc_idx3 = idx[:n_sc].reshape(NW, rows_per_w // 80, 80)
  tc = _segment_sum_tc(attr, idx, num_segments, n_sc)
  sc = _segment_sum_sc(attr, sc_idx3, num_segments, n_sc)
  return sc[0] + sc[1] + tc


# speedup vs baseline: 1.3282x; 1.3282x over previous
import functools

import jax
import jax.numpy as jnp
from jax import lax
from jax.experimental import pallas as pl
from jax.experimental.pallas import tpu as pltpu
from jax.experimental.pallas import tpu_sc as plsc

N_SC_FRAC_NUM, N_SC_FRAC_DEN = 2, 5
N_SC_QUANTUM = 64000
TC_BLK = 4000


def _segment_sum_sc(attr, idx3, num_segments, n_sc):
  N, F = attr.shape
  info = plsc.get_sparse_core_info()
  NC, NS, L = info.num_cores, info.num_subcores, info.num_lanes
  NW = NC * NS
  NF = F // L
  rows_per_w = n_sc // NW
  R = 80
  steps = rows_per_w // R
  NBUF = 4
  NG = R // L

  mesh = plsc.VectorSubcoreMesh(core_axis_name="c", subcore_axis_name="s")

  @functools.partial(
      pl.kernel,
      mesh=mesh,
      out_type=jax.ShapeDtypeStruct((NC, num_segments, F), jnp.float32),
      scratch_types=[
          pltpu.VMEM((steps, R), jnp.int32),
          pltpu.VMEM((NBUF, R, F), jnp.float32),
          pltpu.VMEM((num_segments, F), jnp.float32),
          pltpu.VMEM((num_segments,), jnp.int32),
          pltpu.VMEM_SHARED((num_segments, F), jnp.float32),
          pltpu.SemaphoreType.DMA((NBUF,)),
      ],
  )
  def k(attr_hbm, idx_hbm, out_hbm, idx_all, rows, acc_v, iota_v, acc_sh,
        gsem):
    cid = lax.axis_index("c")
    sid = lax.axis_index("s")
    wid = sid * NC + cid
    base = wid * rows_per_w

    def zrow(i, carry):
      for j in range(NF):
        acc_v[i, pl.ds(j * L, L)] = jnp.zeros((L,), jnp.float32)
      return carry
    lax.fori_loop(0, num_segments, zrow, 0)

    @pl.when(sid == 0)
    def _():
      pltpu.sync_copy(acc_v, acc_sh)

    for kk in range(num_segments // L):
      iota_v[pl.ds(kk * L, L)] = lax.iota(jnp.int32, L) + (kk * L)

    pltpu.sync_copy(idx_hbm.at[wid], idx_all)

    plsc.subcore_barrier()

    def gather(t, bi):
      return pltpu.async_copy(
          attr_hbm.at[pl.ds(base + t * R, R)], rows.at[bi], gsem.at[bi])

    def wait_gather(t, bi):
      pltpu.make_async_copy(
          attr_hbm.at[pl.ds(base + t * R, R)], rows.at[bi], gsem.at[bi]
      ).wait()

    for b in range(NBUF):
      gather(b, b)

    def body(t, carry):
      bi = t & (NBUF - 1)
      wait_gather(t, bi)
      for gi in range(NG):
        a = gi * L
        ids = idx_all[t, pl.ds(a, L)]
        s0 = ids[0]
        s15 = ids[L - 1]

        @pl.when(s0 == s15)
        def _():
          acc = tuple(rows[bi, a, pl.ds(j * L, L)] for j in range(NF))
          for r in range(1, L):
            vals = tuple(rows[bi, a + r, pl.ds(j * L, L)] for j in range(NF))
            acc = tuple(acc[j] + vals[j] for j in range(NF))
          for j in range(NF):
            plsc.addupdate(acc_v.at[s0, pl.ds(j * L, L)], acc[j])

        @pl.when(s0 != s15)
        def _():
          for r in range(L):
            s_r = ids[r]
            for j in range(NF):
              plsc.addupdate(acc_v.at[s_r, pl.ds(j * L, L)],
                             rows[bi, a + r, pl.ds(j * L, L)])

      @pl.when(t + NBUF < steps)
      def _():
        gather(t + NBUF, bi)
      return carry

    lax.fori_loop(0, steps, body, 0)

    pltpu.sync_copy(acc_v, acc_sh.at[iota_v], add=True)

    plsc.subcore_barrier()

    @pl.when(sid == 0)
    def _():
      pltpu.sync_copy(acc_sh, out_hbm.at[cid])

  return k(attr, idx3)


def _segment_sum_tc(attr, idx3, num_segments, n_sc):
  N, F = attr.shape
  nb = (N - n_sc) // TC_BLK
  off = n_sc // TC_BLK

  def body(ids_ref, attr_ref, out_ref):
    i = pl.program_id(0)

    @pl.when(i == 0)
    def _():
      out_ref[...] = jnp.zeros_like(out_ref)

    ids = ids_ref[0]
    iota_s = lax.broadcasted_iota(jnp.int32, (num_segments, TC_BLK), 0)
    onehot = (iota_s == ids).astype(jnp.float32)
    out_ref[...] += jnp.dot(onehot, attr_ref[...],
                            preferred_element_type=jnp.float32)

  return pl.pallas_call(
      body,
      grid=(nb,),
      in_specs=[
          pl.BlockSpec((1, 1, TC_BLK), lambda i: (i + off, 0, 0)),
          pl.BlockSpec((TC_BLK, F), lambda i: (i + off, 0)),
      ],
      out_specs=pl.BlockSpec((num_segments, F), lambda i: (0, 0)),
      out_shape=jax.ShapeDtypeStruct((num_segments, F), jnp.float32),
  )(idx3.reshape(N // TC_BLK, 1, TC_BLK), attr)


def kernel(reference, attr, batch_index):
  num_segments = reference.shape[0]
  N = attr.shape[0]
  n_sc = max(N_SC_QUANTUM,
             (N * N_SC_FRAC_NUM // N_SC_FRAC_DEN // N_SC_QUANTUM)
             * N_SC_QUANTUM)
  idx = batch_index.astype(jnp.int32)
  info = plsc.get_sparse_core_info()
  NW = info.num_cores * info.num_subcores
  rows_per_w = n_sc // NW
  sc_idx3 = idx[:n_sc].reshape(NW, rows_per_w // 80, 80)
  tc = _segment_sum_tc(attr, idx, num_segments, n_sc)
  sc = _segment_sum_sc(attr, sc_idx3, num_segments, n_sc)
  return sc[0] + sc[1] + tc


# speedup vs baseline: 1.4705x; 1.1071x over previous
import functools

import jax
import jax.numpy as jnp
from jax import lax
from jax.experimental import pallas as pl
from jax.experimental.pallas import tpu as pltpu
from jax.experimental.pallas import tpu_sc as plsc

N_SC_FRAC_NUM, N_SC_FRAC_DEN = 2, 5
N_SC_QUANTUM = 64000
TC_BLK = 8000


def _segment_sum_sc(attr, idx3, num_segments, n_sc):
  N, F = attr.shape
  info = plsc.get_sparse_core_info()
  NC, NS, L = info.num_cores, info.num_subcores, info.num_lanes
  NW = NC * NS
  NF = F // L
  rows_per_w = n_sc // NW
  R = 80
  steps = rows_per_w // R
  NBUF = 4
  NG = R // L

  mesh = plsc.VectorSubcoreMesh(core_axis_name="c", subcore_axis_name="s")

  @functools.partial(
      pl.kernel,
      mesh=mesh,
      out_type=jax.ShapeDtypeStruct((NC, num_segments, F), jnp.float32),
      scratch_types=[
          pltpu.VMEM((steps, R), jnp.int32),
          pltpu.VMEM((NBUF, R, F), jnp.float32),
          pltpu.VMEM((num_segments, F), jnp.float32),
          pltpu.VMEM((num_segments,), jnp.int32),
          pltpu.VMEM_SHARED((num_segments, F), jnp.float32),
          pltpu.SemaphoreType.DMA((NBUF,)),
      ],
  )
  def k(attr_hbm, idx_hbm, out_hbm, idx_all, rows, acc_v, iota_v, acc_sh,
        gsem):
    cid = lax.axis_index("c")
    sid = lax.axis_index("s")
    wid = sid * NC + cid
    base = wid * rows_per_w

    def zrow(i, carry):
      for j in range(NF):
        acc_v[i, pl.ds(j * L, L)] = jnp.zeros((L,), jnp.float32)
      return carry
    lax.fori_loop(0, num_segments, zrow, 0)

    @pl.when(sid == 0)
    def _():
      pltpu.sync_copy(acc_v, acc_sh)

    for kk in range(num_segments // L):
      iota_v[pl.ds(kk * L, L)] = lax.iota(jnp.int32, L) + (kk * L)

    pltpu.sync_copy(idx_hbm.at[wid], idx_all)

    plsc.subcore_barrier()

    def gather(t, bi):
      return pltpu.async_copy(
          attr_hbm.at[pl.ds(base + t * R, R)], rows.at[bi], gsem.at[bi])

    def wait_gather(t, bi):
      pltpu.make_async_copy(
          attr_hbm.at[pl.ds(base + t * R, R)], rows.at[bi], gsem.at[bi]
      ).wait()

    for b in range(NBUF):
      gather(b, b)

    def body(t, carry):
      bi = t & (NBUF - 1)
      wait_gather(t, bi)
      for gi in range(NG):
        a = gi * L
        ids = idx_all[t, pl.ds(a, L)]
        s0 = ids[0]
        s15 = ids[L - 1]

        @pl.when(s0 == s15)
        def _():
          acc = tuple(rows[bi, a, pl.ds(j * L, L)] for j in range(NF))
          for r in range(1, L):
            vals = tuple(rows[bi, a + r, pl.ds(j * L, L)] for j in range(NF))
            acc = tuple(acc[j] + vals[j] for j in range(NF))
          for j in range(NF):
            plsc.addupdate(acc_v.at[s0, pl.ds(j * L, L)], acc[j])

        @pl.when(s0 != s15)
        def _():
          for r in range(L):
            s_r = ids[r]
            for j in range(NF):
              plsc.addupdate(acc_v.at[s_r, pl.ds(j * L, L)],
                             rows[bi, a + r, pl.ds(j * L, L)])

      @pl.when(t + NBUF < steps)
      def _():
        gather(t + NBUF, bi)
      return carry

    lax.fori_loop(0, steps, body, 0)

    pltpu.sync_copy(acc_v, acc_sh.at[iota_v], add=True)

    plsc.subcore_barrier()

    @pl.when(sid == 0)
    def _():
      pltpu.sync_copy(acc_sh, out_hbm.at[cid])

  return k(attr, idx3)


def _segment_sum_tc(attr, idx3, num_segments, n_sc):
  N, F = attr.shape
  nb = (N - n_sc) // TC_BLK
  off = n_sc // TC_BLK

  def body(ids_ref, attr_ref, out_ref):
    i = pl.program_id(0)

    @pl.when(i == 0)
    def _():
      out_ref[...] = jnp.zeros_like(out_ref)

    ids = ids_ref[0]
    iota_s = lax.broadcasted_iota(jnp.int32, (num_segments, TC_BLK), 0)
    onehot = (iota_s == ids).astype(jnp.float32)
    out_ref[...] += jnp.dot(onehot, attr_ref[...],
                            preferred_element_type=jnp.float32)

  return pl.pallas_call(
      body,
      grid=(nb,),
      in_specs=[
          pl.BlockSpec((1, 1, TC_BLK), lambda i: (i + off, 0, 0)),
          pl.BlockSpec((TC_BLK, F), lambda i: (i + off, 0)),
      ],
      out_specs=pl.BlockSpec((num_segments, F), lambda i: (0, 0)),
      out_shape=jax.ShapeDtypeStruct((num_segments, F), jnp.float32),
  )(idx3.reshape(N // TC_BLK, 1, TC_BLK), attr)


def kernel(reference, attr, batch_index):
  num_segments = reference.shape[0]
  N = attr.shape[0]
  n_sc = max(N_SC_QUANTUM,
             (N * N_SC_FRAC_NUM // N_SC_FRAC_DEN // N_SC_QUANTUM)
             * N_SC_QUANTUM)
  idx = batch_index.astype(jnp.int32)
  info = plsc.get_sparse_core_info()
  NW = info.num_cores * info.num_subcores
  rows_per_w = n_sc // NW
  sc_idx3 = idx[:n_sc].reshape(NW, rows_per_w // 80, 80)
  tc = _segment_sum_tc(attr, idx, num_segments, n_sc)
  sc = _segment_sum_sc(attr, sc_idx3, num_segments, n_sc)
  return sc[0] + sc[1] + tc


# speedup vs baseline: 1.4866x; 1.0110x over previous
import functools

import jax
import jax.numpy as jnp
from jax import lax
from jax.experimental import pallas as pl
from jax.experimental.pallas import tpu as pltpu
from jax.experimental.pallas import tpu_sc as plsc

N_SC_FRAC_NUM, N_SC_FRAC_DEN = 2, 5
N_SC_QUANTUM = 64000
TC_BLK = 16000


def _segment_sum_sc(attr, idx3, num_segments, n_sc):
  N, F = attr.shape
  info = plsc.get_sparse_core_info()
  NC, NS, L = info.num_cores, info.num_subcores, info.num_lanes
  NW = NC * NS
  NF = F // L
  rows_per_w = n_sc // NW
  R = 80
  steps = rows_per_w // R
  NBUF = 4
  NG = R // L

  mesh = plsc.VectorSubcoreMesh(core_axis_name="c", subcore_axis_name="s")

  @functools.partial(
      pl.kernel,
      mesh=mesh,
      out_type=jax.ShapeDtypeStruct((NC, num_segments, F), jnp.float32),
      scratch_types=[
          pltpu.VMEM((steps, R), jnp.int32),
          pltpu.VMEM((NBUF, R, F), jnp.float32),
          pltpu.VMEM((num_segments, F), jnp.float32),
          pltpu.VMEM((num_segments,), jnp.int32),
          pltpu.VMEM_SHARED((num_segments, F), jnp.float32),
          pltpu.SemaphoreType.DMA((NBUF,)),
      ],
  )
  def k(attr_hbm, idx_hbm, out_hbm, idx_all, rows, acc_v, iota_v, acc_sh,
        gsem):
    cid = lax.axis_index("c")
    sid = lax.axis_index("s")
    wid = sid * NC + cid
    base = wid * rows_per_w

    def zrow(i, carry):
      for j in range(NF):
        acc_v[i, pl.ds(j * L, L)] = jnp.zeros((L,), jnp.float32)
      return carry
    lax.fori_loop(0, num_segments, zrow, 0)

    @pl.when(sid == 0)
    def _():
      pltpu.sync_copy(acc_v, acc_sh)

    for kk in range(num_segments // L):
      iota_v[pl.ds(kk * L, L)] = lax.iota(jnp.int32, L) + (kk * L)

    pltpu.sync_copy(idx_hbm.at[wid], idx_all)

    plsc.subcore_barrier()

    def gather(t, bi):
      return pltpu.async_copy(
          attr_hbm.at[pl.ds(base + t * R, R)], rows.at[bi], gsem.at[bi])

    def wait_gather(t, bi):
      pltpu.make_async_copy(
          attr_hbm.at[pl.ds(base + t * R, R)], rows.at[bi], gsem.at[bi]
      ).wait()

    for b in range(NBUF):
      gather(b, b)

    def body(t, carry):
      bi = t & (NBUF - 1)
      wait_gather(t, bi)
      for gi in range(NG):
        a = gi * L
        ids = idx_all[t, pl.ds(a, L)]
        s0 = ids[0]
        s15 = ids[L - 1]

        @pl.when(s0 == s15)
        def _():
          acc = tuple(rows[bi, a, pl.ds(j * L, L)] for j in range(NF))
          for r in range(1, L):
            vals = tuple(rows[bi, a + r, pl.ds(j * L, L)] for j in range(NF))
            acc = tuple(acc[j] + vals[j] for j in range(NF))
          for j in range(NF):
            plsc.addupdate(acc_v.at[s0, pl.ds(j * L, L)], acc[j])

        @pl.when(s0 != s15)
        def _():
          for r in range(L):
            s_r = ids[r]
            for j in range(NF):
              plsc.addupdate(acc_v.at[s_r, pl.ds(j * L, L)],
                             rows[bi, a + r, pl.ds(j * L, L)])

      @pl.when(t + NBUF < steps)
      def _():
        gather(t + NBUF, bi)
      return carry

    lax.fori_loop(0, steps, body, 0)

    pltpu.sync_copy(acc_v, acc_sh.at[iota_v], add=True)

    plsc.subcore_barrier()

    @pl.when(sid == 0)
    def _():
      pltpu.sync_copy(acc_sh, out_hbm.at[cid])

  return k(attr, idx3)


def _segment_sum_tc(attr, idx3, num_segments, n_sc):
  N, F = attr.shape
  nb = (N - n_sc) // TC_BLK
  off = n_sc // TC_BLK

  def body(ids_ref, attr_ref, out_ref):
    i = pl.program_id(0)

    @pl.when(i == 0)
    def _():
      out_ref[...] = jnp.zeros_like(out_ref)

    ids = ids_ref[0]
    iota_s = lax.broadcasted_iota(jnp.int32, (num_segments, TC_BLK), 0)
    onehot = (iota_s == ids).astype(jnp.float32)
    out_ref[...] += jnp.dot(onehot, attr_ref[...],
                            preferred_element_type=jnp.float32)

  return pl.pallas_call(
      body,
      grid=(nb,),
      in_specs=[
          pl.BlockSpec((1, 1, TC_BLK), lambda i: (i + off, 0, 0)),
          pl.BlockSpec((TC_BLK, F), lambda i: (i + off, 0)),
      ],
      out_specs=pl.BlockSpec((num_segments, F), lambda i: (0, 0)),
      out_shape=jax.ShapeDtypeStruct((num_segments, F), jnp.float32),
  )(idx3.reshape(N // TC_BLK, 1, TC_BLK), attr)


def kernel(reference, attr, batch_index):
  num_segments = reference.shape[0]
  N = attr.shape[0]
  n_sc = max(N_SC_QUANTUM,
             (N * N_SC_FRAC_NUM // N_SC_FRAC_DEN // N_SC_QUANTUM)
             * N_SC_QUANTUM)
  idx = batch_index.astype(jnp.int32)
  info = plsc.get_sparse_core_info()
  NW = info.num_cores * info.num_subcores
  rows_per_w = n_sc // NW
  sc_idx3 = idx[:n_sc].reshape(NW, rows_per_w // 80, 80)
  tc = _segment_sum_tc(attr, idx, num_segments, n_sc)
  sc = _segment_sum_sc(attr, sc_idx3, num_segments, n_sc)
  return sc[0] + sc[1] + tc
